# baseline (speedup 1.0000x reference)
import functools

import jax
import jax.numpy as jnp
from jax import lax
from jax.experimental import pallas as pl
from jax.experimental.pallas import tpu as pltpu
from jax.experimental.pallas import tpu_sc as plsc

_ROWS = 16384
_COLS = 2048

_SC_ROWS = 8192
_TC_ROWS = _ROWS - _SC_ROWS

_NW = 32
_LANES = 16
_T = _SC_ROWS * _COLS // _NW
_C = 32768
_NCH = _T // _C
_UNROLL = 4

_TC_BLOCK = 1024
_QNB = _ROWS // _TC_BLOCK
_TC_NB = _TC_ROWS // _TC_BLOCK


def _sc_minmax_body(x_hbm, omin_hbm, omax_hbm, b0, b1, vmin, vmax, s0, s1):
    wid = lax.axis_index("c") * 16 + lax.axis_index("s")
    base = wid * _T

    def start(chunk, buf, sem):
        pltpu.make_async_copy(
            x_hbm.at[pl.ds(base + chunk * _C, _C)], buf, sem).start()

    def wait(buf, sem):
        pltpu.make_async_copy(x_hbm.at[pl.ds(base, _C)], buf, sem).wait()

    def acc(buf, carry):
        def inner(j, cr):
            mns, mxs = cr
            new_mns, new_mxs = [], []
            for u in range(_UNROLL):
                v = buf[pl.ds(j * (_UNROLL * _LANES) + u * _LANES, _LANES)]
                new_mns.append(jnp.minimum(mns[u], v))
                new_mxs.append(jnp.maximum(mxs[u], v))
            return tuple(new_mns), tuple(new_mxs)

        return lax.fori_loop(0, _C // (_UNROLL * _LANES), inner, carry)

    start(0, b0, s0)
    start(1, b1, s1)

    big = jnp.float32(3.4e38)
    carry0 = (tuple(jnp.full((_LANES,), big, jnp.float32)
                    for _ in range(_UNROLL)),
              tuple(jnp.full((_LANES,), -big, jnp.float32)
                    for _ in range(_UNROLL)))

    def pair_body(p, carry):
        wait(b0, s0)
        carry = acc(b0, carry)

        @pl.when(2 * p + 2 < _NCH)
        def _():
            start(2 * p + 2, b0, s0)

        wait(b1, s1)
        carry = acc(b1, carry)

        @pl.when(2 * p + 3 < _NCH)
        def _():
            start(2 * p + 3, b1, s1)

        return carry

    mns, mxs = lax.fori_loop(0, _NCH // 2, pair_body, carry0)

    vmin[...] = jnp.minimum(jnp.minimum(mns[0], mns[1]),
                            jnp.minimum(mns[2], mns[3]))
    vmax[...] = jnp.maximum(jnp.maximum(mxs[0], mxs[1]),
                            jnp.maximum(mxs[2], mxs[3]))
    pltpu.sync_copy(vmin, omin_hbm.at[wid])
    pltpu.sync_copy(vmax, omax_hbm.at[wid])


def _sc_minmax(x_flat):
    mesh = plsc.VectorSubcoreMesh(core_axis_name="c", subcore_axis_name="s")
    f = pl.kernel(
        _sc_minmax_body,
        mesh=mesh,
        out_type=[jax.ShapeDtypeStruct((_NW, _LANES), jnp.float32),
                  jax.ShapeDtypeStruct((_NW, _LANES), jnp.float32)],
        scratch_types=[pltpu.VMEM((_C,), jnp.float32),
                       pltpu.VMEM((_C,), jnp.float32),
                       pltpu.VMEM((_LANES,), jnp.float32),
                       pltpu.VMEM((_LANES,), jnp.float32),
                       pltpu.SemaphoreType.DMA,
                       pltpu.SemaphoreType.DMA],
    )
    return f(x_flat)


def _tc_mm_body(x_ref, o_ref, mm_ref):
    i = pl.program_id(0)

    @pl.when(i == 0)
    def _():
        mm_ref[0] = jnp.inf
        mm_ref[1] = -jnp.inf

    x = x_ref[...]
    mm_ref[0] = jnp.minimum(mm_ref[0], jnp.min(x))
    mm_ref[1] = jnp.maximum(mm_ref[1], jnp.max(x))

    @pl.when(i == _TC_NB - 1)
    def _():
        o_ref[0] = mm_ref[0]
        o_ref[1] = mm_ref[1]


def _quant_body(nl_ref, tcmm_ref, smin_ref, smax_ref, x_ref, o_ref):
    nl = nl_ref[0]
    mn = jnp.minimum(jnp.min(smin_ref[...]), tcmm_ref[0])
    mx = jnp.maximum(jnp.max(smax_ref[...]), tcmm_ref[1])
    rng = mx - mn
    scale = rng / nl
    inv_scale = nl / rng
    o_ref[...] = jnp.round(x_ref[...] * inv_scale) * scale


def kernel(input, bits):
    nlevels = (jnp.exp2(bits.astype(jnp.float32)) - 1.0
               if hasattr(bits, "astype")
               else jnp.float32(2.0 ** bits - 1.0))
    nlevels = jnp.reshape(nlevels, (1,))
    x2 = input.reshape(_ROWS, _COLS)

    sc_min, sc_max = _sc_minmax(x2[_TC_ROWS:].reshape(-1))

    tc_mm = pl.pallas_call(
        _tc_mm_body,
        grid=(_TC_NB,),
        in_specs=[pl.BlockSpec((_TC_BLOCK, _COLS), lambda i: (i, 0))],
        out_specs=pl.BlockSpec(memory_space=pltpu.SMEM),
        out_shape=jax.ShapeDtypeStruct((2,), jnp.float32),
        scratch_shapes=[pltpu.SMEM((2,), jnp.float32)],
    )(x2[:_TC_ROWS])

    out = pl.pallas_call(
        _quant_body,
        grid=(_QNB,),
        in_specs=[
            pl.BlockSpec(memory_space=pltpu.SMEM),
            pl.BlockSpec(memory_space=pltpu.SMEM),
            pl.BlockSpec((_NW, _LANES), lambda i: (0, 0)),
            pl.BlockSpec((_NW, _LANES), lambda i: (0, 0)),
            pl.BlockSpec((_TC_BLOCK, _COLS), lambda i: (i, 0)),
        ],
        out_specs=pl.BlockSpec((_TC_BLOCK, _COLS), lambda i: (i, 0)),
        out_shape=jax.ShapeDtypeStruct((_ROWS, _COLS), jnp.float32),
    )(nlevels, tc_mm, sc_min, sc_max, x2)
    return out.reshape(input.shape)


# speedup vs baseline: 1.1514x; 1.1514x over previous
import functools

import jax
import jax.numpy as jnp
from jax import lax
from jax.experimental import pallas as pl
from jax.experimental.pallas import tpu as pltpu
from jax.experimental.pallas import tpu_sc as plsc

_ROWS = 16384
_COLS = 2048

_SC_ROWS = 8192
_TC_ROWS = _ROWS - _SC_ROWS

_NW = 32
_LANES = 16
_T = _SC_ROWS * _COLS // _NW
_C = 32768
_NCH = _T // _C
_UNROLL = 4

_TC_BLOCK = 1024
_QNB = _ROWS // _TC_BLOCK
_TC_NB = _TC_ROWS // _TC_BLOCK


def _sc_minmax_body(x_hbm, omin_hbm, omax_hbm, b0, b1, vmin, vmax, s0, s1):
    wid = lax.axis_index("c") * 16 + lax.axis_index("s")
    base = _TC_ROWS * _COLS + wid * _T

    def start(chunk, buf, sem):
        pltpu.make_async_copy(
            x_hbm.at[pl.ds(base + chunk * _C, _C)], buf, sem).start()

    def wait(buf, sem):
        pltpu.make_async_copy(x_hbm.at[pl.ds(base, _C)], buf, sem).wait()

    def acc(buf, carry):
        def inner(j, cr):
            mns, mxs = cr
            new_mns, new_mxs = [], []
            for u in range(_UNROLL):
                v = buf[pl.ds(j * (_UNROLL * _LANES) + u * _LANES, _LANES)]
                new_mns.append(jnp.minimum(mns[u], v))
                new_mxs.append(jnp.maximum(mxs[u], v))
            return tuple(new_mns), tuple(new_mxs)

        return lax.fori_loop(0, _C // (_UNROLL * _LANES), inner, carry)

    start(0, b0, s0)
    start(1, b1, s1)

    big = jnp.float32(3.4e38)
    carry0 = (tuple(jnp.full((_LANES,), big, jnp.float32)
                    for _ in range(_UNROLL)),
              tuple(jnp.full((_LANES,), -big, jnp.float32)
                    for _ in range(_UNROLL)))

    def pair_body(p, carry):
        wait(b0, s0)
        carry = acc(b0, carry)

        @pl.when(2 * p + 2 < _NCH)
        def _():
            start(2 * p + 2, b0, s0)

        wait(b1, s1)
        carry = acc(b1, carry)

        @pl.when(2 * p + 3 < _NCH)
        def _():
            start(2 * p + 3, b1, s1)

        return carry

    mns, mxs = lax.fori_loop(0, _NCH // 2, pair_body, carry0)

    vmin[...] = jnp.minimum(jnp.minimum(mns[0], mns[1]),
                            jnp.minimum(mns[2], mns[3]))
    vmax[...] = jnp.maximum(jnp.maximum(mxs[0], mxs[1]),
                            jnp.maximum(mxs[2], mxs[3]))
    pltpu.sync_copy(vmin, omin_hbm.at[wid])
    pltpu.sync_copy(vmax, omax_hbm.at[wid])


def _sc_minmax(x_flat):
    mesh = plsc.VectorSubcoreMesh(core_axis_name="c", subcore_axis_name="s")
    f = pl.kernel(
        _sc_minmax_body,
        mesh=mesh,
        out_type=[jax.ShapeDtypeStruct((_NW, _LANES), jnp.float32),
                  jax.ShapeDtypeStruct((_NW, _LANES), jnp.float32)],
        scratch_types=[pltpu.VMEM((_C,), jnp.float32),
                       pltpu.VMEM((_C,), jnp.float32),
                       pltpu.VMEM((_LANES,), jnp.float32),
                       pltpu.VMEM((_LANES,), jnp.float32),
                       pltpu.SemaphoreType.DMA,
                       pltpu.SemaphoreType.DMA],
    )
    return f(x_flat)


def _tc_mm_body(x_ref, o_ref, mm_ref):
    i = pl.program_id(0)

    @pl.when(i == 0)
    def _():
        mm_ref[0] = jnp.inf
        mm_ref[1] = -jnp.inf

    x = x_ref[...]
    mm_ref[0] = jnp.minimum(mm_ref[0], jnp.min(x))
    mm_ref[1] = jnp.maximum(mm_ref[1], jnp.max(x))

    @pl.when(i == _TC_NB - 1)
    def _():
        o_ref[0] = mm_ref[0]
        o_ref[1] = mm_ref[1]


def _quant_body(nl_ref, tcmm_ref, smin_ref, smax_ref, x_ref, o_ref):
    nl = nl_ref[0]
    mn = jnp.minimum(jnp.min(smin_ref[...]), tcmm_ref[0])
    mx = jnp.maximum(jnp.max(smax_ref[...]), tcmm_ref[1])
    rng = mx - mn
    scale = rng / nl
    inv_scale = nl / rng
    o_ref[...] = jnp.round(x_ref[...] * inv_scale) * scale


def kernel(input, bits):
    nlevels = (jnp.exp2(bits.astype(jnp.float32)) - 1.0
               if hasattr(bits, "astype")
               else jnp.float32(2.0 ** bits - 1.0))
    nlevels = jnp.reshape(nlevels, (1,))
    x2 = input.reshape(_ROWS, _COLS)

    sc_min, sc_max = _sc_minmax(input.reshape(-1))

    tc_mm = pl.pallas_call(
        _tc_mm_body,
        grid=(_TC_NB,),
        in_specs=[pl.BlockSpec((_TC_BLOCK, _COLS), lambda i: (i, 0))],
        out_specs=pl.BlockSpec(memory_space=pltpu.SMEM),
        out_shape=jax.ShapeDtypeStruct((2,), jnp.float32),
        scratch_shapes=[pltpu.SMEM((2,), jnp.float32)],
    )(x2)

    out = pl.pallas_call(
        _quant_body,
        grid=(_QNB,),
        in_specs=[
            pl.BlockSpec(memory_space=pltpu.SMEM),
            pl.BlockSpec(memory_space=pltpu.SMEM),
            pl.BlockSpec((_NW, _LANES), lambda i: (0, 0)),
            pl.BlockSpec((_NW, _LANES), lambda i: (0, 0)),
            pl.BlockSpec((_TC_BLOCK, _COLS), lambda i: (i, 0)),
        ],
        out_specs=pl.BlockSpec((_TC_BLOCK, _COLS), lambda i: (i, 0)),
        out_shape=jax.ShapeDtypeStruct((_ROWS, _COLS), jnp.float32),
    )(nlevels, tc_mm, sc_min, sc_max, x2)
    return out.reshape(input.shape)


# speedup vs baseline: 1.8753x; 1.6286x over previous
import functools

import jax
import jax.numpy as jnp
from jax import lax
from jax.experimental import pallas as pl
from jax.experimental.pallas import tpu as pltpu
from jax.experimental.pallas import tpu_sc as plsc

_ROWS = 16384
_COLS = 2048

_SC_ROWS = 8192
_TC_ROWS = _ROWS - _SC_ROWS

_NW = 32
_LANES = 16
_T = _SC_ROWS * _COLS // _NW
_C = 32768
_NCH = _T // _C
_UNROLL = 4

_TC_BLOCK = 1024
_QNB = _ROWS // _TC_BLOCK
_TC_NB = _TC_ROWS // _TC_BLOCK


def _sc_minmax_body(x_hbm, omin_hbm, omax_hbm, b0, b1, vmin, vmax, s0, s1):
    wid = lax.axis_index("c") * 16 + lax.axis_index("s")
    rows_per_w = _T // _COLS
    crows = _C // _COLS
    base = _TC_ROWS + wid * rows_per_w

    def start(chunk, buf, sem):
        pltpu.make_async_copy(
            x_hbm.at[pl.ds(base + chunk * crows, crows)], buf, sem).start()

    def wait(buf, sem):
        pltpu.make_async_copy(x_hbm.at[pl.ds(base, crows)], buf, sem).wait()

    def acc(buf, carry):
        def row_body(r, cr):
            def inner(j, cr2):
                mns, mxs = cr2
                new_mns, new_mxs = [], []
                for u in range(_UNROLL):
                    v = buf[r, pl.ds(j * (_UNROLL * _LANES) + u * _LANES,
                                     _LANES)]
                    new_mns.append(jnp.minimum(mns[u], v))
                    new_mxs.append(jnp.maximum(mxs[u], v))
                return tuple(new_mns), tuple(new_mxs)

            return lax.fori_loop(0, _COLS // (_UNROLL * _LANES), inner, cr)

        return lax.fori_loop(0, crows, row_body, carry)

    start(0, b0, s0)
    start(1, b1, s1)

    big = jnp.float32(3.4e38)
    carry0 = (tuple(jnp.full((_LANES,), big, jnp.float32)
                    for _ in range(_UNROLL)),
              tuple(jnp.full((_LANES,), -big, jnp.float32)
                    for _ in range(_UNROLL)))

    def pair_body(p, carry):
        wait(b0, s0)
        carry = acc(b0, carry)

        @pl.when(2 * p + 2 < _NCH)
        def _():
            start(2 * p + 2, b0, s0)

        wait(b1, s1)
        carry = acc(b1, carry)

        @pl.when(2 * p + 3 < _NCH)
        def _():
            start(2 * p + 3, b1, s1)

        return carry

    mns, mxs = lax.fori_loop(0, _NCH // 2, pair_body, carry0)

    vmin[...] = jnp.minimum(jnp.minimum(mns[0], mns[1]),
                            jnp.minimum(mns[2], mns[3]))
    vmax[...] = jnp.maximum(jnp.maximum(mxs[0], mxs[1]),
                            jnp.maximum(mxs[2], mxs[3]))
    pltpu.sync_copy(vmin, omin_hbm.at[wid])
    pltpu.sync_copy(vmax, omax_hbm.at[wid])


def _sc_minmax(x_flat):
    mesh = plsc.VectorSubcoreMesh(core_axis_name="c", subcore_axis_name="s")
    f = pl.kernel(
        _sc_minmax_body,
        mesh=mesh,
        out_type=[jax.ShapeDtypeStruct((_NW, _LANES), jnp.float32),
                  jax.ShapeDtypeStruct((_NW, _LANES), jnp.float32)],
        scratch_types=[pltpu.VMEM((_C // _COLS, _COLS), jnp.float32),
                       pltpu.VMEM((_C // _COLS, _COLS), jnp.float32),
                       pltpu.VMEM((_LANES,), jnp.float32),
                       pltpu.VMEM((_LANES,), jnp.float32),
                       pltpu.SemaphoreType.DMA,
                       pltpu.SemaphoreType.DMA],
    )
    return f(x_flat)


def _tc_mm_body(x_ref, o_ref, mm_ref):
    i = pl.program_id(0)

    @pl.when(i == 0)
    def _():
        mm_ref[0] = jnp.inf
        mm_ref[1] = -jnp.inf

    x = x_ref[...]
    mm_ref[0] = jnp.minimum(mm_ref[0], jnp.min(x))
    mm_ref[1] = jnp.maximum(mm_ref[1], jnp.max(x))

    @pl.when(i == _TC_NB - 1)
    def _():
        o_ref[0] = mm_ref[0]
        o_ref[1] = mm_ref[1]


def _quant_body(nl_ref, tcmm_ref, smin_ref, smax_ref, x_ref, o_ref):
    nl = nl_ref[0]
    mn = jnp.minimum(jnp.min(smin_ref[...]), tcmm_ref[0])
    mx = jnp.maximum(jnp.max(smax_ref[...]), tcmm_ref[1])
    rng = mx - mn
    scale = rng / nl
    inv_scale = nl / rng
    o_ref[...] = jnp.round(x_ref[...] * inv_scale) * scale


def kernel(input, bits):
    nlevels = (jnp.exp2(bits.astype(jnp.float32)) - 1.0
               if hasattr(bits, "astype")
               else jnp.float32(2.0 ** bits - 1.0))
    nlevels = jnp.reshape(nlevels, (1,))
    x2 = input.reshape(_ROWS, _COLS)

    sc_min, sc_max = _sc_minmax(x2)

    tc_mm = pl.pallas_call(
        _tc_mm_body,
        grid=(_TC_NB,),
        in_specs=[pl.BlockSpec((_TC_BLOCK, _COLS), lambda i: (i, 0))],
        out_specs=pl.BlockSpec(memory_space=pltpu.SMEM),
        out_shape=jax.ShapeDtypeStruct((2,), jnp.float32),
        scratch_shapes=[pltpu.SMEM((2,), jnp.float32)],
    )(x2)

    out = pl.pallas_call(
        _quant_body,
        grid=(_QNB,),
        in_specs=[
            pl.BlockSpec(memory_space=pltpu.SMEM),
            pl.BlockSpec(memory_space=pltpu.SMEM),
            pl.BlockSpec((_NW, _LANES), lambda i: (0, 0)),
            pl.BlockSpec((_NW, _LANES), lambda i: (0, 0)),
            pl.BlockSpec((_TC_BLOCK, _COLS), lambda i: (i, 0)),
        ],
        out_specs=pl.BlockSpec((_TC_BLOCK, _COLS), lambda i: (i, 0)),
        out_shape=jax.ShapeDtypeStruct((_ROWS, _COLS), jnp.float32),
    )(nlevels, tc_mm, sc_min, sc_max, x2)
    return out.reshape(input.shape)


# speedup vs baseline: 1.9066x; 1.0167x over previous
import functools

import jax
import jax.numpy as jnp
from jax import lax
from jax.experimental import pallas as pl
from jax.experimental.pallas import tpu as pltpu
from jax.experimental.pallas import tpu_sc as plsc

_ROWS = 16384
_COLS = 2048

_SC_ROWS = 5120
_TC_ROWS = _ROWS - _SC_ROWS

_NW = 32
_LANES = 16
_T = _SC_ROWS * _COLS // _NW
_C = 32768
_NCH = _T // _C
_UNROLL = 8

_TC_BLOCK = 1024
_QNB = _ROWS // _TC_BLOCK
_TC_NB = _TC_ROWS // _TC_BLOCK


def _sc_minmax_body(x_hbm, omin_hbm, omax_hbm, b0, b1, vmin, vmax, s0, s1):
    wid = lax.axis_index("c") * 16 + lax.axis_index("s")
    rows_per_w = _T // _COLS
    crows = _C // _COLS
    base = _TC_ROWS + wid * rows_per_w

    def start(chunk, buf, sem):
        pltpu.make_async_copy(
            x_hbm.at[pl.ds(base + chunk * crows, crows)], buf, sem).start()

    def wait(buf, sem):
        pltpu.make_async_copy(x_hbm.at[pl.ds(base, crows)], buf, sem).wait()

    def acc(buf, carry):
        def row_body(r, cr):
            def inner(j, cr2):
                mns, mxs = cr2
                new_mns, new_mxs = [], []
                for u in range(_UNROLL):
                    v = buf[r, pl.ds(j * (_UNROLL * _LANES) + u * _LANES,
                                     _LANES)]
                    new_mns.append(jnp.minimum(mns[u], v))
                    new_mxs.append(jnp.maximum(mxs[u], v))
                return tuple(new_mns), tuple(new_mxs)

            return lax.fori_loop(0, _COLS // (_UNROLL * _LANES), inner, cr)

        return lax.fori_loop(0, crows, row_body, carry)

    start(0, b0, s0)
    start(1, b1, s1)

    big = jnp.float32(3.4e38)
    carry0 = (tuple(jnp.full((_LANES,), big, jnp.float32)
                    for _ in range(_UNROLL)),
              tuple(jnp.full((_LANES,), -big, jnp.float32)
                    for _ in range(_UNROLL)))

    def pair_body(p, carry):
        wait(b0, s0)
        carry = acc(b0, carry)

        @pl.when(2 * p + 2 < _NCH)
        def _():
            start(2 * p + 2, b0, s0)

        wait(b1, s1)
        carry = acc(b1, carry)

        @pl.when(2 * p + 3 < _NCH)
        def _():
            start(2 * p + 3, b1, s1)

        return carry

    mns, mxs = lax.fori_loop(0, _NCH // 2, pair_body, carry0)

    vmin[...] = functools.reduce(jnp.minimum, mns)
    vmax[...] = functools.reduce(jnp.maximum, mxs)
    pltpu.sync_copy(vmin, omin_hbm.at[wid])
    pltpu.sync_copy(vmax, omax_hbm.at[wid])


def _sc_minmax(x_flat):
    mesh = plsc.VectorSubcoreMesh(core_axis_name="c", subcore_axis_name="s")
    f = pl.kernel(
        _sc_minmax_body,
        mesh=mesh,
        out_type=[jax.ShapeDtypeStruct((_NW, _LANES), jnp.float32),
                  jax.ShapeDtypeStruct((_NW, _LANES), jnp.float32)],
        scratch_types=[pltpu.VMEM((_C // _COLS, _COLS), jnp.float32),
                       pltpu.VMEM((_C // _COLS, _COLS), jnp.float32),
                       pltpu.VMEM((_LANES,), jnp.float32),
                       pltpu.VMEM((_LANES,), jnp.float32),
                       pltpu.SemaphoreType.DMA,
                       pltpu.SemaphoreType.DMA],
    )
    return f(x_flat)


def _tc_mm_body(x_ref, o_ref, mm_ref):
    i = pl.program_id(0)

    @pl.when(i == 0)
    def _():
        mm_ref[0] = jnp.inf
        mm_ref[1] = -jnp.inf

    x = x_ref[...]
    mm_ref[0] = jnp.minimum(mm_ref[0], jnp.min(x))
    mm_ref[1] = jnp.maximum(mm_ref[1], jnp.max(x))

    @pl.when(i == _TC_NB - 1)
    def _():
        o_ref[0] = mm_ref[0]
        o_ref[1] = mm_ref[1]


def _quant_body(nl_ref, tcmm_ref, smin_ref, smax_ref, x_ref, o_ref):
    nl = nl_ref[0]
    mn = jnp.minimum(jnp.min(smin_ref[...]), tcmm_ref[0])
    mx = jnp.maximum(jnp.max(smax_ref[...]), tcmm_ref[1])
    rng = mx - mn
    scale = rng / nl
    inv_scale = nl / rng
    o_ref[...] = jnp.round(x_ref[...] * inv_scale) * scale


def kernel(input, bits):
    nlevels = (jnp.exp2(bits.astype(jnp.float32)) - 1.0
               if hasattr(bits, "astype")
               else jnp.float32(2.0 ** bits - 1.0))
    nlevels = jnp.reshape(nlevels, (1,))
    x2 = input.reshape(_ROWS, _COLS)

    sc_min, sc_max = _sc_minmax(x2)

    tc_mm = pl.pallas_call(
        _tc_mm_body,
        grid=(_TC_NB,),
        in_specs=[pl.BlockSpec((_TC_BLOCK, _COLS), lambda i: (i, 0))],
        out_specs=pl.BlockSpec(memory_space=pltpu.SMEM),
        out_shape=jax.ShapeDtypeStruct((2,), jnp.float32),
        scratch_shapes=[pltpu.SMEM((2,), jnp.float32)],
    )(x2)

    out = pl.pallas_call(
        _quant_body,
        grid=(_QNB,),
        in_specs=[
            pl.BlockSpec(memory_space=pltpu.SMEM),
            pl.BlockSpec(memory_space=pltpu.SMEM),
            pl.BlockSpec((_NW, _LANES), lambda i: (0, 0)),
            pl.BlockSpec((_NW, _LANES), lambda i: (0, 0)),
            pl.BlockSpec((_TC_BLOCK, _COLS), lambda i: (i, 0)),
        ],
        out_specs=pl.BlockSpec((_TC_BLOCK, _COLS), lambda i: (i, 0)),
        out_shape=jax.ShapeDtypeStruct((_ROWS, _COLS), jnp.float32),
    )(nlevels, tc_mm, sc_min, sc_max, x2)
    return out.reshape(input.shape)


# speedup vs baseline: 1.9528x; 1.0242x over previous
import functools

import jax
import jax.numpy as jnp
from jax import lax
from jax.experimental import pallas as pl
from jax.experimental.pallas import tpu as pltpu
from jax.experimental.pallas import tpu_sc as plsc

_ROWS = 16384
_COLS = 2048

_SC_ROWS = 4096
_TC_ROWS = _ROWS - _SC_ROWS

_NW = 32
_LANES = 16
_T = _SC_ROWS * _COLS // _NW
_C = 32768
_NCH = _T // _C
_UNROLL = 8

_TC_BLOCK = 1024
_QNB = _ROWS // _TC_BLOCK
_TC_NB = _TC_ROWS // _TC_BLOCK


def _sc_minmax_body(x_hbm, omin_hbm, omax_hbm, b0, b1, vmin, vmax, s0, s1):
    wid = lax.axis_index("c") * 16 + lax.axis_index("s")
    rows_per_w = _T // _COLS
    crows = _C // _COLS
    base = _TC_ROWS + wid * rows_per_w

    def start(chunk, buf, sem):
        pltpu.make_async_copy(
            x_hbm.at[pl.ds(base + chunk * crows, crows)], buf, sem).start()

    def wait(buf, sem):
        pltpu.make_async_copy(x_hbm.at[pl.ds(base, crows)], buf, sem).wait()

    def acc(buf, carry):
        def row_body(r, cr):
            def inner(j, cr2):
                mns, mxs = cr2
                new_mns, new_mxs = [], []
                for u in range(_UNROLL):
                    v = buf[r, pl.ds(j * (_UNROLL * _LANES) + u * _LANES,
                                     _LANES)]
                    new_mns.append(jnp.minimum(mns[u], v))
                    new_mxs.append(jnp.maximum(mxs[u], v))
                return tuple(new_mns), tuple(new_mxs)

            return lax.fori_loop(0, _COLS // (_UNROLL * _LANES), inner, cr)

        return lax.fori_loop(0, crows, row_body, carry)

    start(0, b0, s0)
    start(1, b1, s1)

    big = jnp.float32(3.4e38)
    carry0 = (tuple(jnp.full((_LANES,), big, jnp.float32)
                    for _ in range(_UNROLL)),
              tuple(jnp.full((_LANES,), -big, jnp.float32)
                    for _ in range(_UNROLL)))

    def pair_body(p, carry):
        wait(b0, s0)
        carry = acc(b0, carry)

        @pl.when(2 * p + 2 < _NCH)
        def _():
            start(2 * p + 2, b0, s0)

        wait(b1, s1)
        carry = acc(b1, carry)

        @pl.when(2 * p + 3 < _NCH)
        def _():
            start(2 * p + 3, b1, s1)

        return carry

    mns, mxs = lax.fori_loop(0, _NCH // 2, pair_body, carry0)

    vmin[...] = functools.reduce(jnp.minimum, mns)
    vmax[...] = functools.reduce(jnp.maximum, mxs)
    pltpu.sync_copy(vmin, omin_hbm.at[wid])
    pltpu.sync_copy(vmax, omax_hbm.at[wid])


def _sc_minmax(x_flat):
    mesh = plsc.VectorSubcoreMesh(core_axis_name="c", subcore_axis_name="s")
    f = pl.kernel(
        _sc_minmax_body,
        mesh=mesh,
        out_type=[jax.ShapeDtypeStruct((_NW, _LANES), jnp.float32),
                  jax.ShapeDtypeStruct((_NW, _LANES), jnp.float32)],
        scratch_types=[pltpu.VMEM((_C // _COLS, _COLS), jnp.float32),
                       pltpu.VMEM((_C // _COLS, _COLS), jnp.float32),
                       pltpu.VMEM((_LANES,), jnp.float32),
                       pltpu.VMEM((_LANES,), jnp.float32),
                       pltpu.SemaphoreType.DMA,
                       pltpu.SemaphoreType.DMA],
    )
    return f(x_flat)


def _tc_mm_body(x_ref, o_ref, accmin_ref, accmax_ref):
    i = pl.program_id(0)

    @pl.when(i == 0)
    def _():
        accmin_ref[...] = jnp.full((8, _COLS), 3.4e38, jnp.float32)
        accmax_ref[...] = jnp.full((8, _COLS), -3.4e38, jnp.float32)

    x = x_ref[...]
    mn = accmin_ref[...]
    mx = accmax_ref[...]
    for u in range(_TC_BLOCK // 8):
        s = x[u * 8:(u + 1) * 8, :]
        mn = jnp.minimum(mn, s)
        mx = jnp.maximum(mx, s)
    accmin_ref[...] = mn
    accmax_ref[...] = mx

    @pl.when(i == _TC_NB - 1)
    def _():
        o_ref[0] = jnp.min(mn)
        o_ref[1] = jnp.max(mx)


def _quant_body(nl_ref, tcmm_ref, smin_ref, smax_ref, x_ref, o_ref):
    nl = nl_ref[0]
    mn = jnp.minimum(jnp.min(smin_ref[...]), tcmm_ref[0])
    mx = jnp.maximum(jnp.max(smax_ref[...]), tcmm_ref[1])
    rng = mx - mn
    scale = rng / nl
    inv_scale = nl / rng
    o_ref[...] = jnp.round(x_ref[...] * inv_scale) * scale


def kernel(input, bits):
    nlevels = (jnp.exp2(bits.astype(jnp.float32)) - 1.0
               if hasattr(bits, "astype")
               else jnp.float32(2.0 ** bits - 1.0))
    nlevels = jnp.reshape(nlevels, (1,))
    x2 = input.reshape(_ROWS, _COLS)

    sc_min, sc_max = _sc_minmax(x2)

    tc_mm = pl.pallas_call(
        _tc_mm_body,
        grid=(_TC_NB,),
        in_specs=[pl.BlockSpec((_TC_BLOCK, _COLS), lambda i: (i, 0))],
        out_specs=pl.BlockSpec(memory_space=pltpu.SMEM),
        out_shape=jax.ShapeDtypeStruct((2,), jnp.float32),
        scratch_shapes=[pltpu.VMEM((8, _COLS), jnp.float32),
                        pltpu.VMEM((8, _COLS), jnp.float32)],
    )(x2)

    out = pl.pallas_call(
        _quant_body,
        grid=(_QNB,),
        in_specs=[
            pl.BlockSpec(memory_space=pltpu.SMEM),
            pl.BlockSpec(memory_space=pltpu.SMEM),
            pl.BlockSpec((_NW, _LANES), lambda i: (0, 0)),
            pl.BlockSpec((_NW, _LANES), lambda i: (0, 0)),
            pl.BlockSpec((_TC_BLOCK, _COLS), lambda i: (i, 0)),
        ],
        out_specs=pl.BlockSpec((_TC_BLOCK, _COLS), lambda i: (i, 0)),
        out_shape=jax.ShapeDtypeStruct((_ROWS, _COLS), jnp.float32),
    )(nlevels, tc_mm, sc_min, sc_max, x2)
    return out.reshape(input.shape)


# speedup vs baseline: 2.2547x; 1.1546x over previous
import jax
import jax.numpy as jnp
from jax.experimental import pallas as pl
from jax.experimental.pallas import tpu as pltpu

_ROWS = 16384
_COLS = 2048
_BLOCK_ROWS = 1024
_NB = _ROWS // _BLOCK_ROWS


def _quant_body(nl_ref, x_ref, o_ref, accmin_ref, accmax_ref, mm_ref):
    p = pl.program_id(0)
    i = pl.program_id(1)

    @pl.when(p == 0)
    def _reduce_phase():
        @pl.when(i == 0)
        def _init():
            accmin_ref[...] = jnp.full((8, _COLS), 3.4e38, jnp.float32)
            accmax_ref[...] = jnp.full((8, _COLS), -3.4e38, jnp.float32)

        x = x_ref[...]
        mn = accmin_ref[...]
        mx = accmax_ref[...]
        for u in range(_BLOCK_ROWS // 8):
            s = x[u * 8:(u + 1) * 8, :]
            mn = jnp.minimum(mn, s)
            mx = jnp.maximum(mx, s)
        accmin_ref[...] = mn
        accmax_ref[...] = mx

    @pl.when(p == 1)
    def _quantize_phase():
        @pl.when(i == 0)
        def _finalize():
            mm_ref[0] = jnp.min(accmin_ref[...])
            mm_ref[1] = jnp.max(accmax_ref[...])

        nl = nl_ref[0]
        rng = mm_ref[1] - mm_ref[0]
        scale = rng / nl
        inv_scale = nl / rng
        o_ref[...] = jnp.round(x_ref[...] * inv_scale) * scale


def kernel(input, bits):
    nlevels = (jnp.exp2(bits.astype(jnp.float32)) - 1.0
               if hasattr(bits, "astype")
               else jnp.float32(2.0 ** bits - 1.0))
    nlevels = jnp.reshape(nlevels, (1,))
    x2 = input.reshape(_ROWS, _COLS)
    out = pl.pallas_call(
        _quant_body,
        grid=(2, _NB),
        in_specs=[
            pl.BlockSpec(memory_space=pltpu.SMEM),
            pl.BlockSpec((_BLOCK_ROWS, _COLS), lambda p, i: (i, 0)),
        ],
        out_specs=pl.BlockSpec((_BLOCK_ROWS, _COLS), lambda p, i: (p * i, 0)),
        out_shape=jax.ShapeDtypeStruct((_ROWS, _COLS), jnp.float32),
        scratch_shapes=[pltpu.VMEM((8, _COLS), jnp.float32),
                        pltpu.VMEM((8, _COLS), jnp.float32),
                        pltpu.SMEM((2,), jnp.float32)],
    )(nlevels, x2)
    return out.reshape(input.shape)


# speedup vs baseline: 2.3077x; 1.0235x over previous
import jax
import jax.numpy as jnp
from jax.experimental import pallas as pl
from jax.experimental.pallas import tpu as pltpu

_ROWS = 16384
_COLS = 2048
_BLOCK_ROWS = 512
_NB = _ROWS // _BLOCK_ROWS
_K = 9


def _quant_body(nl_ref, x_ref, o_ref, res_ref, accmin_ref, accmax_ref,
                mm_ref):
    p = pl.program_id(0)
    i = pl.program_id(1)

    @pl.when(p == 0)
    def _reduce_phase():
        @pl.when(i == 0)
        def _init():
            accmin_ref[...] = jnp.full((8, _COLS), 3.4e38, jnp.float32)
            accmax_ref[...] = jnp.full((8, _COLS), -3.4e38, jnp.float32)

        x = x_ref[...]
        mn = accmin_ref[...]
        mx = accmax_ref[...]
        for u in range(_BLOCK_ROWS // 8):
            s = x[u * 8:(u + 1) * 8, :]
            mn = jnp.minimum(mn, s)
            mx = jnp.maximum(mx, s)
        accmin_ref[...] = mn
        accmax_ref[...] = mx

        @pl.when(i < _K)
        def _stash():
            res_ref[pl.ds(i * _BLOCK_ROWS, _BLOCK_ROWS), :] = x

    @pl.when(p == 1)
    def _quantize_phase():
        @pl.when(i == 0)
        def _finalize():
            mm_ref[0] = jnp.min(accmin_ref[...])
            mm_ref[1] = jnp.max(accmax_ref[...])

        nl = nl_ref[0]
        rng = mm_ref[1] - mm_ref[0]
        scale = rng / nl
        inv_scale = nl / rng

        @pl.when(i < _K)
        def _from_vmem():
            r = res_ref[pl.ds(i * _BLOCK_ROWS, _BLOCK_ROWS), :]
            o_ref[...] = jnp.round(r * inv_scale) * scale

        @pl.when(i >= _K)
        def _from_hbm():
            o_ref[...] = jnp.round(x_ref[...] * inv_scale) * scale


def kernel(input, bits):
    nlevels = (jnp.exp2(bits.astype(jnp.float32)) - 1.0
               if hasattr(bits, "astype")
               else jnp.float32(2.0 ** bits - 1.0))
    nlevels = jnp.reshape(nlevels, (1,))
    x2 = input.reshape(_ROWS, _COLS)

    def x_map(p, i):
        return (jnp.where(p == 0, i, jnp.where(i < _K, _NB - 1, i)), 0)

    out = pl.pallas_call(
        _quant_body,
        grid=(2, _NB),
        in_specs=[
            pl.BlockSpec(memory_space=pltpu.SMEM),
            pl.BlockSpec((_BLOCK_ROWS, _COLS), x_map),
        ],
        out_specs=pl.BlockSpec((_BLOCK_ROWS, _COLS), lambda p, i: (p * i, 0)),
        out_shape=jax.ShapeDtypeStruct((_ROWS, _COLS), jnp.float32),
        scratch_shapes=[pltpu.VMEM((_K * _BLOCK_ROWS, _COLS), jnp.float32),
                        pltpu.VMEM((8, _COLS), jnp.float32),
                        pltpu.VMEM((8, _COLS), jnp.float32),
                        pltpu.SMEM((2,), jnp.float32)],
    )(nlevels, x2)
    return out.reshape(input.shape)
